# Initial kernel scaffold; baseline (speedup 1.0000x reference)
import jax
import jax.numpy as jnp
from jax.experimental import pallas as pl


def kernel(x, edge_index, Wp, bp, g0, b0, Wl, bl, Wr, bng, bnb, skipW, skipb, W1, b1, W2, b2, W3, b3):
    raise NotImplementedError("write your pallas kernel here")


# speedup vs baseline: 4.3812x; 4.3812x over previous
import functools

import jax
import jax.numpy as jnp
from jax import lax
from jax.experimental import pallas as pl
from jax.experimental.pallas import tpu as pltpu
from jax.experimental.pallas import tpu_sc as plsc

N = 10000
E = 320000
H = 128
HH = H // 2
L = 6

NC = 2
NS = 16
NW = NC * NS
CH = 100
NCH = (E // NS) // CH
CCH = 80
CNCH = (E // NW) // CCH
RZ = 80
NZ = N // RZ
NZT = -(-NZ // NS)
CNT_W = 64


def _zero_buf(buf, rows, width):
    per_row = width // 16

    def body(t, _):
        buf[t // per_row, pl.ds((t % per_row) * 16, 16)] = jnp.zeros((16,), jnp.float32)
        return _

    lax.fori_loop(0, rows * per_row, body, None)


@functools.cache
def _make_sc_kernels():
    mesh = plsc.VectorSubcoreMesh(core_axis_name="c", subcore_axis_name="s",
                                  num_cores=NC, num_subcores=NS)
    params = pltpu.CompilerParams(use_tc_tiling_on_sc=False)
    agg = pl.kernel(
        _sc_aggregate,
        out_type=jax.ShapeDtypeStruct((NC, N, HH), jnp.float32),
        mesh=mesh,
        compiler_params=params,
        scratch_types=[
            pltpu.VMEM((NCH, CH), jnp.int32),
            pltpu.VMEM((NCH, CH), jnp.int32),
            pltpu.VMEM((CH, HH), jnp.float32),
            pltpu.VMEM((CH, HH), jnp.float32),
            pltpu.VMEM_SHARED((N, HH), jnp.float32),
        ],
    )
    cnt = pl.kernel(
        _sc_count,
        out_type=jax.ShapeDtypeStruct((NC, N, CNT_W), jnp.float32),
        mesh=mesh,
        scratch_types=[
            pltpu.VMEM((CNCH, CCH), jnp.int32),
            pltpu.VMEM((CCH, CNT_W), jnp.float32),
            pltpu.VMEM_SHARED((N, CNT_W), jnp.float32),
        ],
    )
    return agg, cnt


def _sc_aggregate(h_hbm, src_hbm, dst_hbm, out_hbm, src_v, dst_v, buf0, buf1, acc):
    c = lax.axis_index("c")
    s = lax.axis_index("s")
    pltpu.sync_copy(src_hbm.at[s], src_v)
    pltpu.sync_copy(dst_hbm.at[s], dst_v)

    _zero_buf(buf0, CH, HH)

    def zrow(t, _):
        k = s + t * NS

        @pl.when(k < NZ)
        def _do():
            pltpu.sync_copy(buf0.at[pl.ds(0, RZ)], acc.at[pl.ds(k * RZ, RZ)])

        return _

    lax.fori_loop(0, NZT, zrow, None)
    plsc.subcore_barrier()

    def chunk(j, _):
        pltpu.sync_copy(h_hbm.at[c].at[src_v.at[j]], buf0)
        pltpu.sync_copy(buf0, acc.at[dst_v.at[j]], add=True)
        return _

    lax.fori_loop(0, NCH, chunk, None)
    plsc.subcore_barrier()

    def orow(t, _):
        k = s + t * NS

        @pl.when(k < NZ)
        def _do():
            pltpu.sync_copy(acc.at[pl.ds(k * RZ, RZ)], buf1.at[pl.ds(0, RZ)])
            pltpu.sync_copy(buf1.at[pl.ds(0, RZ)], out_hbm.at[c, pl.ds(k * RZ, RZ)])

        return _

    lax.fori_loop(0, NZT, orow, None)


def _sc_count(dst_hbm, out_hbm, dst_v, ones_v, acc):
    c = lax.axis_index("c")
    s = lax.axis_index("s")
    w = c * NS + s
    pltpu.sync_copy(dst_hbm.at[w], dst_v)

    _zero_buf(ones_v, CCH, CNT_W)

    def zrow(t, _):
        k = s + t * NS

        @pl.when(k < NZ)
        def _do():
            pltpu.sync_copy(ones_v, acc.at[pl.ds(k * RZ, RZ)])

        return _

    lax.fori_loop(0, NZT, zrow, None)
    plsc.subcore_barrier()

    def fill(t, _):
        ones_v[t, pl.ds(0, 16)] = jnp.ones((16,), jnp.float32)
        return _

    lax.fori_loop(0, CCH, fill, None)

    def chunk(j, _):
        pltpu.sync_copy(ones_v, acc.at[dst_v.at[j]], add=True)
        return _

    lax.fori_loop(0, CNCH, chunk, None)
    plsc.subcore_barrier()

    def orow(t, _):
        k = s + t * NS

        @pl.when(k < NZ)
        def _do():
            pltpu.sync_copy(acc.at[pl.ds(k * RZ, RZ)], ones_v)
            pltpu.sync_copy(ones_v, out_hbm.at[c, pl.ds(k * RZ, RZ)])

        return _

    lax.fori_loop(0, NZT, orow, None)


def _hp_dot(a, b):
    return jnp.dot(a, b, preferred_element_type=jnp.float32)


NB = 5
BR = N // NB


def _stats_body(z_ref, st_ref):
    z = z_ref[...]
    m = jnp.mean(z, axis=0, keepdims=True)
    v = jnp.mean((z - m) * (z - m), axis=0, keepdims=True)
    st_ref[...] = jnp.concatenate([m, v], axis=0)


def _tc_stats(z):
    return pl.pallas_call(
        _stats_body,
        out_shape=jax.ShapeDtypeStruct((2, H), jnp.float32),
    )(z)


def _finish_bn(z, st, g, b):
    m = st[0:1, :]
    v = st[1:2, :]
    return jnp.maximum((z - m) / jnp.sqrt(v + 1e-5) * g + b, 0.0)


def _split(z, out_ref):
    out_ref[0] = z[:, :HH]
    out_ref[1] = z[:, HH:]


def _proj_a_body(x_ref, wp_ref, bp_ref, z_ref):
    z_ref[...] = _hp_dot(x_ref[...], wp_ref[...].T) + bp_ref[...]


def _proj_b_body(z_ref, st_ref, g_ref, b_ref, cnt_ref, h2_ref, invc_ref):
    h = _finish_bn(z_ref[...], st_ref[...], g_ref[...], b_ref[...])
    _split(h, h2_ref)
    invc_ref[...] = jnp.maximum(cnt_ref[0, :, 0:1], 1.0)


def _tc_proj(x, wp, bp, g0, b0, cnt16):
    z = pl.pallas_call(
        _proj_a_body,
        grid=(NB,),
        in_specs=[
            pl.BlockSpec((BR, H), lambda i: (i, 0)),
            pl.BlockSpec((H, H), lambda i: (0, 0)),
            pl.BlockSpec((1, H), lambda i: (0, 0)),
        ],
        out_specs=pl.BlockSpec((BR, H), lambda i: (i, 0)),
        out_shape=jax.ShapeDtypeStruct((N, H), jnp.float32),
    )(x, wp, bp)
    st = _tc_stats(z)
    return pl.pallas_call(
        _proj_b_body,
        grid=(NB,),
        in_specs=[
            pl.BlockSpec((BR, H), lambda i: (i, 0)),
            pl.BlockSpec((2, H), lambda i: (0, 0)),
            pl.BlockSpec((1, H), lambda i: (0, 0)),
            pl.BlockSpec((1, H), lambda i: (0, 0)),
            pl.BlockSpec((NC, BR, CNT_W), lambda i: (0, i, 0)),
        ],
        out_specs=(
            pl.BlockSpec((NC, BR, HH), lambda i: (0, i, 0)),
            pl.BlockSpec((BR, 1), lambda i: (i, 0)),
        ),
        out_shape=(
            jax.ShapeDtypeStruct((NC, N, HH), jnp.float32),
            jax.ShapeDtypeStruct((N, 1), jnp.float32),
        ),
    )(z, st, g0, b0, cnt16)


def _layer_a_body(h2_ref, agg_ref, invc_ref, wl_ref, bl_ref, wr_ref, z_ref):
    h = jnp.concatenate([h2_ref[0], h2_ref[1]], axis=-1)
    mean = jnp.concatenate([agg_ref[0], agg_ref[1]], axis=-1) / invc_ref[...]
    z = _hp_dot(mean, wl_ref[...].T) + bl_ref[...]
    z_ref[...] = z + _hp_dot(h, wr_ref[...].T)


def _layer_b_body(has_skip, z_ref, st_ref, g_ref, b_ref, *rest):
    if has_skip:
        h2_ref, skw_ref, skb_ref, out_ref = rest
    else:
        (out_ref,) = rest
    z = _finish_bn(z_ref[...], st_ref[...], g_ref[...], b_ref[...])
    if has_skip:
        h = jnp.concatenate([h2_ref[0], h2_ref[1]], axis=-1)
        z = z + _hp_dot(h, skw_ref[...].T) + skb_ref[...]
    _split(z, out_ref)


def _tc_layer(h2, agg, invc, wl, bl, wr, g, b, skw=None, skb=None):
    has_skip = skw is not None
    z = pl.pallas_call(
        _layer_a_body,
        grid=(NB,),
        in_specs=[
            pl.BlockSpec((NC, BR, HH), lambda i: (0, i, 0)),
            pl.BlockSpec((NC, BR, HH), lambda i: (0, i, 0)),
            pl.BlockSpec((BR, 1), lambda i: (i, 0)),
            pl.BlockSpec((H, H), lambda i: (0, 0)),
            pl.BlockSpec((1, H), lambda i: (0, 0)),
            pl.BlockSpec((H, H), lambda i: (0, 0)),
        ],
        out_specs=pl.BlockSpec((BR, H), lambda i: (i, 0)),
        out_shape=jax.ShapeDtypeStruct((N, H), jnp.float32),
    )(h2, agg, invc, wl, bl, wr)
    st = _tc_stats(z)
    in_specs = [
        pl.BlockSpec((BR, H), lambda i: (i, 0)),
        pl.BlockSpec((2, H), lambda i: (0, 0)),
        pl.BlockSpec((1, H), lambda i: (0, 0)),
        pl.BlockSpec((1, H), lambda i: (0, 0)),
    ]
    args = (z, st, g, b)
    if has_skip:
        in_specs += [
            pl.BlockSpec((NC, BR, HH), lambda i: (0, i, 0)),
            pl.BlockSpec((H, H), lambda i: (0, 0)),
            pl.BlockSpec((1, H), lambda i: (0, 0)),
        ]
        args += (h2, skw, skb)
    return pl.pallas_call(
        functools.partial(_layer_b_body, has_skip),
        grid=(NB,),
        in_specs=in_specs,
        out_specs=pl.BlockSpec((NC, BR, HH), lambda i: (0, i, 0)),
        out_shape=jax.ShapeDtypeStruct((NC, N, HH), jnp.float32),
    )(*args)


def _head_body(h2_ref, w1_ref, b1_ref, w2_ref, b2_ref, w3_ref, b3_ref,
               o_ref, emb_ref):
    h = jnp.concatenate([h2_ref[0], h2_ref[1]], axis=-1)
    emb_ref[...] = h
    z = jnp.maximum(_hp_dot(h, w1_ref[...].T) + b1_ref[...], 0.0)
    z = jnp.maximum(_hp_dot(z, w2_ref[...].T) + b2_ref[...], 0.0)
    o_ref[...] = _hp_dot(z, w3_ref[...].T) + b3_ref[...]


def _tc_head(h2, w1, b1, w2, b2, w3, b3):
    return pl.pallas_call(
        _head_body,
        grid=(NB,),
        in_specs=[
            pl.BlockSpec((NC, BR, HH), lambda i: (0, i, 0)),
            pl.BlockSpec((H // 2, H), lambda i: (0, 0)),
            pl.BlockSpec((1, H // 2), lambda i: (0, 0)),
            pl.BlockSpec((H // 4, H // 2), lambda i: (0, 0)),
            pl.BlockSpec((1, H // 4), lambda i: (0, 0)),
            pl.BlockSpec((2, H // 4), lambda i: (0, 0)),
            pl.BlockSpec((1, 2), lambda i: (0, 0)),
        ],
        out_specs=(
            pl.BlockSpec((BR, 2), lambda i: (i, 0)),
            pl.BlockSpec((BR, H), lambda i: (i, 0)),
        ),
        out_shape=(
            jax.ShapeDtypeStruct((N, 2), jnp.float32),
            jax.ShapeDtypeStruct((N, H), jnp.float32),
        ),
    )(h2, w1, b1, w2, b2, w3, b3)


def kernel(x, edge_index, Wp, bp, g0, b0, Wl, bl, Wr, bng, bnb, skipW, skipb,
           W1, b1, W2, b2, W3, b3):
    src16 = edge_index[0].reshape(NS, NCH, CH)
    dst16 = edge_index[1].reshape(NS, NCH, CH)

    sc_aggregate, _ = _make_sc_kernels()
    cnt16 = sc_aggregate(jnp.ones((NC, N, HH), jnp.float32), src16, dst16)
    h2, invc = _tc_proj(x, Wp, bp.reshape(1, H), g0.reshape(1, H),
                        b0.reshape(1, H), cnt16)

    skip_idx = 0
    for i in range(L):
        agg = sc_aggregate(h2, src16, dst16)
        if (i + 1) % 2 == 0:
            h2 = _tc_layer(h2, agg, invc, Wl[i], bl[i].reshape(1, H), Wr[i],
                           bng[i].reshape(1, H), bnb[i].reshape(1, H),
                           skipW[skip_idx], skipb[skip_idx].reshape(1, H))
            skip_idx += 1
        else:
            h2 = _tc_layer(h2, agg, invc, Wl[i], bl[i].reshape(1, H), Wr[i],
                           bng[i].reshape(1, H), bnb[i].reshape(1, H))

    o, emb = _tc_head(h2, W1, b1.reshape(1, H // 2), W2, b2.reshape(1, H // 4),
                      W3, b3.reshape(1, 2))
    return (o, emb)
